# baseline (speedup 1.0000x reference)
import functools

import jax
import jax.numpy as jnp
from jax import lax
from jax.experimental import pallas as pl
from jax.experimental.pallas import tpu as pltpu
from jax.experimental.pallas import tpu_sc as plsc

VOCAB = 100000
EMB_DIM = 16
PACK = EMB_DIM // 2
OUT_DIM = 10
BATCH = 16384
EPS = 1e-5


def _make_sc_gather(batch, dim, vocab):
    info = plsc.get_sparse_core_info()
    nc, ns = info.num_cores, info.num_subcores
    nw = nc * ns
    assert batch % (16 * nw) == 0
    bw = batch // nw
    mesh = plsc.VectorSubcoreMesh(core_axis_name="c", subcore_axis_name="s")

    @functools.partial(
        pl.kernel,
        mesh=mesh,
        out_type=jax.ShapeDtypeStruct((dim, batch), jnp.int32),
        scratch_types=[
            pltpu.VMEM_SHARED((dim, vocab), jnp.int32),
            pltpu.VMEM((bw,), jnp.int32),
            pltpu.VMEM((dim, bw), jnp.int32),
            pltpu.SemaphoreType.DMA,
            pltpu.SemaphoreType.DMA,
        ],
        compiler_params=pltpu.CompilerParams(use_tc_tiling_on_sc=False),
    )
    def gather_kernel(table_hbm, idx_hbm, out_hbm, shared_t, idx_v, stg,
                      sem_t, sem):
        sid = lax.axis_index("s")
        wid = sid * nc + lax.axis_index("c")
        base = wid * bw
        half = vocab // 2
        row = sid // 2
        col = (sid % 2) * half
        pltpu.async_copy(table_hbm.at[pl.ds(row, 1), pl.ds(col, half)],
                         shared_t.at[pl.ds(row, 1), pl.ds(col, half)],
                         sem_t)
        pltpu.sync_copy(idx_hbm.at[pl.ds(base, bw)], idx_v)
        pltpu.make_async_copy(
            table_hbm.at[pl.ds(row, 1), pl.ds(col, half)],
            shared_t.at[pl.ds(row, 1), pl.ds(col, half)], sem_t).wait()
        plsc.subcore_barrier()

        def fire(g):
            v = idx_v[pl.ds(g * 16, 16)]
            for j in range(16):
                pltpu.async_copy(
                    shared_t.at[:, pl.ds(v[j], 1)],
                    stg.at[:, pl.ds(g * 16 + j, 1)], sem)

        def drain(g):
            for j in range(16):
                pltpu.make_async_copy(
                    table_hbm.at[:, pl.ds(0, 1)],
                    stg.at[:, pl.ds(g * 16 + j, 1)], sem).wait()

        def body(g, carry):
            fire(g)

            @pl.when(g > 0)
            def _():
                drain(g - 1)

            return carry

        nch = bw // 16
        lax.fori_loop(0, nch, body, 0)
        drain(nch - 1)
        pltpu.sync_copy(stg, out_hbm.at[:, pl.ds(base, bw)])

    return gather_kernel


def _tc_tail_body(ages_ref, feats_ref, gamma_ref, beta_ref, we_ref, wo_ref,
                  wa_ref, b_ref, out_ref):
    a = ages_ref[...]
    n = a.shape[1]
    mean = jnp.sum(a) / n
    centered = a - mean
    var = jnp.sum(centered * centered) / n
    a_hat = centered * lax.rsqrt(var + EPS)
    age_feat = a_hat * gamma_ref[0] + beta_ref[0]
    age_relu = jnp.maximum(age_feat, 0.0)
    packed = feats_ref[...]
    f_even = lax.bitcast_convert_type(
        lax.shift_left(packed, 16), jnp.float32)
    f_odd = lax.bitcast_convert_type(
        packed & jnp.int32(-65536), jnp.float32)
    f_even = jnp.maximum(f_even, 0.0)
    f_odd = jnp.maximum(f_odd, 0.0)
    out = lax.dot_general(
        we_ref[...], f_even, (((0,), (0,)), ((), ())),
        preferred_element_type=jnp.float32)
    out = out + lax.dot_general(
        wo_ref[...], f_odd, (((0,), (0,)), ((), ())),
        preferred_element_type=jnp.float32)
    out = out + wa_ref[...] * age_relu
    out = out + b_ref[...]
    out_ref[...] = out


def kernel(customer_ids, ages, emb_table, bn_gamma, bn_beta, W, b):
    table_t = emb_table.T
    tb = lax.bitcast_convert_type(
        table_t.astype(jnp.bfloat16), jnp.uint16).astype(jnp.uint32)
    packed = (tb[0::2, :] | (tb[1::2, :] << 16)).astype(jnp.int32)
    packed_flat = packed.reshape(PACK * VOCAB)
    packed = packed_flat.reshape(PACK, VOCAB)
    feats_p = _make_sc_gather(BATCH, PACK, VOCAB)(
        packed, customer_ids.astype(jnp.int32))
    out_t = pl.pallas_call(
        _tc_tail_body,
        out_shape=jax.ShapeDtypeStruct((OUT_DIM, BATCH), jnp.float32),
        in_specs=[
            pl.BlockSpec(memory_space=pltpu.VMEM),
            pl.BlockSpec(memory_space=pltpu.VMEM),
            pl.BlockSpec(memory_space=pltpu.SMEM),
            pl.BlockSpec(memory_space=pltpu.SMEM),
            pl.BlockSpec(memory_space=pltpu.VMEM),
            pl.BlockSpec(memory_space=pltpu.VMEM),
            pl.BlockSpec(memory_space=pltpu.VMEM),
            pl.BlockSpec(memory_space=pltpu.VMEM),
        ],
        out_specs=pl.BlockSpec(memory_space=pltpu.VMEM),
    )(ages.reshape(1, BATCH), feats_p, bn_gamma, bn_beta,
      W[0:EMB_DIM:2, :], W[1:EMB_DIM:2, :], W[EMB_DIM].reshape(OUT_DIM, 1),
      b.reshape(OUT_DIM, 1))
    return out_t.T


# speedup vs baseline: 1.0002x; 1.0002x over previous
import functools

import jax
import jax.numpy as jnp
from jax import lax
from jax.experimental import pallas as pl
from jax.experimental.pallas import tpu as pltpu
from jax.experimental.pallas import tpu_sc as plsc

VOCAB = 100000
EMB_DIM = 16
PACK = EMB_DIM // 2
OUT_DIM = 10
BATCH = 16384
EPS = 1e-5


def _make_sc_gather(batch, dim, vocab):
    info = plsc.get_sparse_core_info()
    nc, ns = info.num_cores, info.num_subcores
    nw = nc * ns
    assert batch % (16 * nw) == 0
    bw = batch // nw
    mesh = plsc.VectorSubcoreMesh(core_axis_name="c", subcore_axis_name="s")

    @functools.partial(
        pl.kernel,
        mesh=mesh,
        out_type=jax.ShapeDtypeStruct((dim, batch), jnp.int32),
        scratch_types=[
            pltpu.VMEM_SHARED((dim, vocab), jnp.int32),
            pltpu.VMEM((bw,), jnp.int32),
            pltpu.VMEM((dim, bw), jnp.int32),
            pltpu.SemaphoreType.DMA,
            pltpu.SemaphoreType.DMA,
        ],
        compiler_params=pltpu.CompilerParams(use_tc_tiling_on_sc=False),
    )
    def gather_kernel(table_hbm, idx_hbm, out_hbm, shared_t, idx_v, stg,
                      sem_t, sem):
        sid = lax.axis_index("s")
        wid = sid * nc + lax.axis_index("c")
        base = wid * bw
        half = vocab // 2
        row = sid // 2
        col = (sid % 2) * half
        pltpu.async_copy(table_hbm.at[pl.ds(row, 1), pl.ds(col, half)],
                         shared_t.at[pl.ds(row, 1), pl.ds(col, half)],
                         sem_t)
        pltpu.sync_copy(idx_hbm.at[pl.ds(base, bw)], idx_v)
        pltpu.make_async_copy(
            table_hbm.at[pl.ds(row, 1), pl.ds(col, half)],
            shared_t.at[pl.ds(row, 1), pl.ds(col, half)], sem_t).wait()
        plsc.subcore_barrier()

        def fire(g):
            v = idx_v[pl.ds(g * 16, 16)]
            for j in range(16):
                pltpu.async_copy(
                    shared_t.at[:, pl.ds(v[j], 1)],
                    stg.at[:, pl.ds(g * 16 + j, 1)], sem)

        def drain(g):
            for j in range(16):
                pltpu.make_async_copy(
                    table_hbm.at[:, pl.ds(0, 1)],
                    stg.at[:, pl.ds(g * 16 + j, 1)], sem).wait()

        def body(g, carry):
            fire(g)

            @pl.when(g > 0)
            def _():
                drain(g - 1)

            return carry

        nch = bw // 16
        lax.fori_loop(0, nch, body, 0)
        drain(nch - 1)
        pltpu.sync_copy(stg, out_hbm.at[:, pl.ds(base, bw)])

    return gather_kernel


def _tc_tail_body(ages_ref, feats_ref, gamma_ref, beta_ref, we_ref, wo_ref,
                  wa_ref, b_ref, out_ref):
    a = ages_ref[...]
    n = a.shape[1]
    mean = jnp.sum(a) / n
    centered = a - mean
    var = jnp.sum(centered * centered) / n
    a_hat = centered * lax.rsqrt(var + EPS)
    age_feat = a_hat * gamma_ref[0] + beta_ref[0]
    age_relu = jnp.maximum(age_feat, 0.0)
    packed = feats_ref[...]
    f_even = lax.bitcast_convert_type(
        lax.shift_left(packed, 16), jnp.float32)
    f_odd = lax.bitcast_convert_type(
        packed & jnp.int32(-65536), jnp.float32)
    f_even = jnp.maximum(f_even, 0.0)
    f_odd = jnp.maximum(f_odd, 0.0)
    out = lax.dot_general(
        we_ref[...], f_even, (((0,), (0,)), ((), ())),
        preferred_element_type=jnp.float32)
    out = out + lax.dot_general(
        wo_ref[...], f_odd, (((0,), (0,)), ((), ())),
        preferred_element_type=jnp.float32)
    out = out + wa_ref[...] * age_relu
    out = out + b_ref[...]
    out_ref[...] = out


def kernel(customer_ids, ages, emb_table, bn_gamma, bn_beta, W, b):
    tb = lax.bitcast_convert_type(
        emb_table.astype(jnp.bfloat16), jnp.uint16).astype(jnp.uint32)
    packed = (tb[:, 0::2] | (tb[:, 1::2] << 16)).astype(jnp.int32)
    packed = packed.T
    feats_p = _make_sc_gather(BATCH, PACK, VOCAB)(
        packed, customer_ids.astype(jnp.int32))
    out_t = pl.pallas_call(
        _tc_tail_body,
        out_shape=jax.ShapeDtypeStruct((OUT_DIM, BATCH), jnp.float32),
        in_specs=[
            pl.BlockSpec(memory_space=pltpu.VMEM),
            pl.BlockSpec(memory_space=pltpu.VMEM),
            pl.BlockSpec(memory_space=pltpu.SMEM),
            pl.BlockSpec(memory_space=pltpu.SMEM),
            pl.BlockSpec(memory_space=pltpu.VMEM),
            pl.BlockSpec(memory_space=pltpu.VMEM),
            pl.BlockSpec(memory_space=pltpu.VMEM),
            pl.BlockSpec(memory_space=pltpu.VMEM),
        ],
        out_specs=pl.BlockSpec(memory_space=pltpu.VMEM),
    )(ages.reshape(1, BATCH), feats_p, bn_gamma, bn_beta,
      W[0:EMB_DIM:2, :], W[1:EMB_DIM:2, :], W[EMB_DIM].reshape(OUT_DIM, 1),
      b.reshape(OUT_DIM, 1))
    return out_t.T


# speedup vs baseline: 1.9662x; 1.9658x over previous
import functools

import jax
import jax.numpy as jnp
from jax import lax
from jax.experimental import pallas as pl
from jax.experimental.pallas import tpu as pltpu
from jax.experimental.pallas import tpu_sc as plsc

VOCAB = 100000
EMB_DIM = 16
PACK = EMB_DIM // 2
OUT_DIM = 10
BATCH = 16384
EPS = 1e-5


def _make_sc_gather(batch, dim, vocab):
    info = plsc.get_sparse_core_info()
    nc, ns = info.num_cores, info.num_subcores
    nw = nc * ns
    assert batch % (16 * nw) == 0
    bw = batch // nw
    mesh = plsc.VectorSubcoreMesh(core_axis_name="c", subcore_axis_name="s")

    @functools.partial(
        pl.kernel,
        mesh=mesh,
        out_type=jax.ShapeDtypeStruct((dim, batch), jnp.int32),
        scratch_types=[
            pltpu.VMEM_SHARED((dim, vocab), jnp.int32),
            pltpu.VMEM((bw,), jnp.int32),
            pltpu.VMEM((dim, bw), jnp.int32),
            pltpu.SemaphoreType.DMA,
            pltpu.SemaphoreType.DMA,
        ],
        compiler_params=pltpu.CompilerParams(use_tc_tiling_on_sc=False),
    )
    def gather_kernel(table_hbm, idx_hbm, out_hbm, shared_t, idx_v, stg,
                      sem_t, sem):
        sid = lax.axis_index("s")
        wid = sid * nc + lax.axis_index("c")
        base = wid * bw
        half = vocab // 2
        row = sid // 2
        col = (sid % 2) * half
        pltpu.async_copy(table_hbm.at[pl.ds(row, 1), pl.ds(col, half)],
                         shared_t.at[pl.ds(row, 1), pl.ds(col, half)],
                         sem_t)
        pltpu.sync_copy(idx_hbm.at[pl.ds(base, bw)], idx_v)
        pltpu.make_async_copy(
            table_hbm.at[pl.ds(row, 1), pl.ds(col, half)],
            shared_t.at[pl.ds(row, 1), pl.ds(col, half)], sem_t).wait()
        plsc.subcore_barrier()

        def fire(g):
            v = idx_v[pl.ds(g * 16, 16)]
            for j in range(16):
                pltpu.async_copy(
                    shared_t.at[:, pl.ds(v[j], 1)],
                    stg.at[:, pl.ds(g * 16 + j, 1)], sem)

        def drain(g):
            for j in range(16):
                pltpu.make_async_copy(
                    table_hbm.at[:, pl.ds(0, 1)],
                    stg.at[:, pl.ds(g * 16 + j, 1)], sem).wait()

        def body(g, carry):
            fire(g)

            @pl.when(g > 0)
            def _():
                drain(g - 1)

            return carry

        nch = bw // 16
        lax.fori_loop(0, nch, body, 0)
        drain(nch - 1)
        pltpu.sync_copy(stg, out_hbm.at[:, pl.ds(base, bw)])

    return gather_kernel


def _tc_tail_body(ages_ref, feats_ref, gamma_ref, beta_ref, we_ref, wo_ref,
                  wa_ref, b_ref, out_ref):
    a = ages_ref[...]
    n = a.shape[1]
    mean = jnp.sum(a) / n
    centered = a - mean
    var = jnp.sum(centered * centered) / n
    a_hat = centered * lax.rsqrt(var + EPS)
    age_feat = a_hat * gamma_ref[0] + beta_ref[0]
    age_relu = jnp.maximum(age_feat, 0.0)
    packed = feats_ref[...]
    f_even = lax.bitcast_convert_type(
        lax.shift_left(packed, 16), jnp.float32)
    f_odd = lax.bitcast_convert_type(
        packed & jnp.int32(-65536), jnp.float32)
    f_even = jnp.maximum(f_even, 0.0)
    f_odd = jnp.maximum(f_odd, 0.0)
    out = lax.dot_general(
        we_ref[...], f_even, (((0,), (0,)), ((), ())),
        preferred_element_type=jnp.float32)
    out = out + lax.dot_general(
        wo_ref[...], f_odd, (((0,), (0,)), ((), ())),
        preferred_element_type=jnp.float32)
    out = out + wa_ref[...] * age_relu
    out = out + b_ref[...]
    out_ref[...] = out


def kernel(customer_ids, ages, emb_table, bn_gamma, bn_beta, W, b):
    u = lax.bitcast_convert_type(emb_table, jnp.uint32)
    r = (u + jnp.uint32(0x7FFF) + ((u >> 16) & jnp.uint32(1))) >> 16
    packed = (r[:, :PACK] | (r[:, PACK:] << 16)).astype(jnp.int32)
    packed = packed.T
    feats_p = _make_sc_gather(BATCH, PACK, VOCAB)(
        packed, customer_ids.astype(jnp.int32))
    out_t = pl.pallas_call(
        _tc_tail_body,
        out_shape=jax.ShapeDtypeStruct((OUT_DIM, BATCH), jnp.float32),
        in_specs=[
            pl.BlockSpec(memory_space=pltpu.VMEM),
            pl.BlockSpec(memory_space=pltpu.VMEM),
            pl.BlockSpec(memory_space=pltpu.SMEM),
            pl.BlockSpec(memory_space=pltpu.SMEM),
            pl.BlockSpec(memory_space=pltpu.VMEM),
            pl.BlockSpec(memory_space=pltpu.VMEM),
            pl.BlockSpec(memory_space=pltpu.VMEM),
            pl.BlockSpec(memory_space=pltpu.VMEM),
        ],
        out_specs=pl.BlockSpec(memory_space=pltpu.VMEM),
    )(ages.reshape(1, BATCH), feats_p, bn_gamma, bn_beta,
      W[0:PACK, :], W[PACK:EMB_DIM, :], W[EMB_DIM].reshape(OUT_DIM, 1),
      b.reshape(OUT_DIM, 1))
    return out_t.T


# speedup vs baseline: 1.9735x; 1.0037x over previous
import functools

import jax
import jax.numpy as jnp
from jax import lax
from jax.experimental import pallas as pl
from jax.experimental.pallas import tpu as pltpu
from jax.experimental.pallas import tpu_sc as plsc

VOCAB = 100000
EMB_DIM = 16
PACK = EMB_DIM // 2
OUT_DIM = 10
BATCH = 16384
EPS = 1e-5


def _make_sc_gather(batch, dim, vocab):
    info = plsc.get_sparse_core_info()
    nc, ns = info.num_cores, info.num_subcores
    nw = nc * ns
    assert batch % (16 * nw) == 0
    bw = batch // nw
    mesh = plsc.VectorSubcoreMesh(core_axis_name="c", subcore_axis_name="s")

    @functools.partial(
        pl.kernel,
        mesh=mesh,
        out_type=jax.ShapeDtypeStruct((dim, batch), jnp.int32),
        scratch_types=[
            pltpu.VMEM_SHARED((dim, vocab), jnp.int32),
            pltpu.VMEM((bw,), jnp.int32),
            pltpu.VMEM((dim, bw), jnp.int32),
            pltpu.SemaphoreType.DMA,
            pltpu.SemaphoreType.DMA,
        ],
        compiler_params=pltpu.CompilerParams(use_tc_tiling_on_sc=False),
    )
    def gather_kernel(table_hbm, idx_hbm, out_hbm, shared_t, idx_v, stg,
                      sem_t, sem):
        sid = lax.axis_index("s")
        wid = sid * nc + lax.axis_index("c")
        base = wid * bw
        half = vocab // 2
        row = sid // 2
        col = (sid % 2) * half
        pltpu.async_copy(table_hbm.at[pl.ds(row, 1), pl.ds(col, half)],
                         shared_t.at[pl.ds(row, 1), pl.ds(col, half)],
                         sem_t)
        pltpu.sync_copy(idx_hbm.at[pl.ds(base, bw)], idx_v)
        pltpu.make_async_copy(
            table_hbm.at[pl.ds(row, 1), pl.ds(col, half)],
            shared_t.at[pl.ds(row, 1), pl.ds(col, half)], sem_t).wait()
        plsc.subcore_barrier()

        def fire(g):
            v = idx_v[pl.ds(g * 16, 16)]
            for j in range(16):
                pltpu.async_copy(
                    shared_t.at[:, pl.ds(v[j], 1)],
                    stg.at[:, pl.ds(g * 16 + j, 1)], sem)

        def drain(g):
            for j in range(16):
                pltpu.make_async_copy(
                    table_hbm.at[:, pl.ds(0, 1)],
                    stg.at[:, pl.ds(g * 16 + j, 1)], sem).wait()

        def body(g, carry):
            fire(g)

            @pl.when(g > 1)
            def _():
                drain(g - 2)

            return carry

        nch = bw // 16
        lax.fori_loop(0, nch, body, 0)
        drain(nch - 2)
        drain(nch - 1)
        pltpu.sync_copy(stg, out_hbm.at[:, pl.ds(base, bw)])

    return gather_kernel


def _tc_tail_body(ages_ref, feats_ref, gamma_ref, beta_ref, we_ref, wo_ref,
                  wa_ref, b_ref, out_ref):
    a = ages_ref[...]
    n = a.shape[1]
    mean = jnp.sum(a) / n
    centered = a - mean
    var = jnp.sum(centered * centered) / n
    a_hat = centered * lax.rsqrt(var + EPS)
    age_feat = a_hat * gamma_ref[0] + beta_ref[0]
    age_relu = jnp.maximum(age_feat, 0.0)
    packed = feats_ref[...]
    f_even = lax.bitcast_convert_type(
        lax.shift_left(packed, 16), jnp.float32)
    f_odd = lax.bitcast_convert_type(
        packed & jnp.int32(-65536), jnp.float32)
    f_even = jnp.maximum(f_even, 0.0)
    f_odd = jnp.maximum(f_odd, 0.0)
    out = lax.dot_general(
        we_ref[...], f_even, (((0,), (0,)), ((), ())),
        preferred_element_type=jnp.float32)
    out = out + lax.dot_general(
        wo_ref[...], f_odd, (((0,), (0,)), ((), ())),
        preferred_element_type=jnp.float32)
    out = out + wa_ref[...] * age_relu
    out = out + b_ref[...]
    out_ref[...] = out


def kernel(customer_ids, ages, emb_table, bn_gamma, bn_beta, W, b):
    u = lax.bitcast_convert_type(emb_table, jnp.uint32)
    r = (u + jnp.uint32(0x7FFF) + ((u >> 16) & jnp.uint32(1))) >> 16
    packed = (r[:, :PACK] | (r[:, PACK:] << 16)).astype(jnp.int32)
    packed = packed.T
    feats_p = _make_sc_gather(BATCH, PACK, VOCAB)(
        packed, customer_ids.astype(jnp.int32))
    out_t = pl.pallas_call(
        _tc_tail_body,
        out_shape=jax.ShapeDtypeStruct((OUT_DIM, BATCH), jnp.float32),
        in_specs=[
            pl.BlockSpec(memory_space=pltpu.VMEM),
            pl.BlockSpec(memory_space=pltpu.VMEM),
            pl.BlockSpec(memory_space=pltpu.SMEM),
            pl.BlockSpec(memory_space=pltpu.SMEM),
            pl.BlockSpec(memory_space=pltpu.VMEM),
            pl.BlockSpec(memory_space=pltpu.VMEM),
            pl.BlockSpec(memory_space=pltpu.VMEM),
            pl.BlockSpec(memory_space=pltpu.VMEM),
        ],
        out_specs=pl.BlockSpec(memory_space=pltpu.VMEM),
    )(ages.reshape(1, BATCH), feats_p, bn_gamma, bn_beta,
      W[0:PACK, :], W[PACK:EMB_DIM, :], W[EMB_DIM].reshape(OUT_DIM, 1),
      b.reshape(OUT_DIM, 1))
    return out_t.T
